# baseline (speedup 1.0000x reference)
import functools

import jax
import jax.numpy as jnp
from jax import lax
from jax.experimental import pallas as pl
from jax.experimental.pallas import tpu as pltpu
from jax.experimental.pallas import tpu_sc as plsc

L1, L2, ORBIT, DIM = 6, 4, 1, 2
NSITES = L1 * L2 * ORBIT
BATCH = 16384

NUM_CORES = 2
NUM_SUBCORES = 16
NUM_WORKERS = NUM_CORES * NUM_SUBCORES
LANES = 16
BW = BATCH // NUM_WORKERS
NCHUNK = BW // LANES
NSTREAM = BW // 128


def _sc_kernel(x_hbm, wave_hbm, out_hbm, xv, idxv, outv, sem):
    wid = lax.axis_index("s") * NUM_CORES + lax.axis_index("c")

    pltpu.sync_copy(x_hbm.at[wid], xv)

    lane = lax.iota(jnp.int32, LANES) * NSITES

    def chunk(c, carry):
        row0 = c * (LANES * NSITES)
        acc = plsc.load_gather(xv, [lane + row0])
        for i in range(1, NSITES):
            g = plsc.load_gather(xv, [lane + (row0 + i)])
            acc = acc + acc + g
        idxv[pl.ds(c * LANES, LANES)] = acc
        return carry

    lax.fori_loop(0, NCHUNK, chunk, 0)

    copies = [
        pltpu.async_copy(
            wave_hbm.at[idxv.at[pl.ds(j * 128, 128)]], outv.at[j], sem
        )
        for j in range(NSTREAM)
    ]
    for c in copies:
        c.wait()

    pltpu.sync_copy(outv, out_hbm.at[pl.ds(wid * NSTREAM, NSTREAM)])


@jax.jit
def _run(xf, wave):
    mesh = plsc.VectorSubcoreMesh(core_axis_name="c", subcore_axis_name="s")
    grid = functools.partial(
        pl.kernel,
        out_type=jax.ShapeDtypeStruct((BATCH // 128, 128), jnp.float32),
        mesh=mesh,
        scratch_types=[
            pltpu.VMEM((BW * NSITES,), jnp.int32),
            pltpu.VMEM((BW,), jnp.int32),
            pltpu.VMEM((NSTREAM, 128), jnp.float32),
            pltpu.SemaphoreType.DMA,
        ],
        compiler_params=pltpu.CompilerParams(needs_layout_passes=False),
    )
    return grid(_sc_kernel)(xf, wave)


def kernel(x, wave):
    xf = x.reshape(NUM_WORKERS, BW * NSITES).astype(jnp.int32)
    return _run(xf, wave).reshape(x.shape[:-3])


# speedup vs baseline: 2.3931x; 2.3931x over previous
import functools

import jax
import jax.numpy as jnp
from jax import lax
from jax.experimental import pallas as pl
from jax.experimental.pallas import tpu as pltpu
from jax.experimental.pallas import tpu_sc as plsc

L1, L2, ORBIT, DIM = 6, 4, 1, 2
NSITES = L1 * L2 * ORBIT
BATCH = 16384

NUM_CORES = 2
NUM_SUBCORES = 16
NUM_WORKERS = NUM_CORES * NUM_SUBCORES
BW = BATCH // NUM_WORKERS
NSTREAM = BW // 128

PACK_BLOCK = 2048
STRIDES = tuple(DIM ** (NSITES - 1 - i) for i in range(NSITES))


def _pack_kernel(x_ref, idx_ref):
    i = lax.broadcasted_iota(jnp.int32, (1, NSITES), 1)
    w = jnp.left_shift(1, (NSITES - 1) - i)
    idx_ref[...] = jnp.sum(x_ref[...] * w, axis=1)


def _sc_gather(idx_hbm, wave_hbm, out_hbm, idxv, outv, sem):
    wid = lax.axis_index("s") * NUM_CORES + lax.axis_index("c")

    pltpu.sync_copy(idx_hbm.at[pl.ds(wid * BW, BW)], idxv)

    copies = [
        pltpu.async_copy(
            wave_hbm.at[idxv.at[pl.ds(j * 128, 128)]], outv.at[j], sem
        )
        for j in range(NSTREAM)
    ]
    for c in copies:
        c.wait()

    pltpu.sync_copy(outv, out_hbm.at[pl.ds(wid * NSTREAM, NSTREAM)])


@jax.jit
def _run(x2, wave):
    idx = pl.pallas_call(
        _pack_kernel,
        grid=(BATCH // PACK_BLOCK,),
        in_specs=[pl.BlockSpec((PACK_BLOCK, NSITES), lambda b: (b, 0))],
        out_specs=pl.BlockSpec((PACK_BLOCK,), lambda b: (b,)),
        out_shape=jax.ShapeDtypeStruct((BATCH,), jnp.int32),
    )(x2)

    mesh = plsc.VectorSubcoreMesh(core_axis_name="c", subcore_axis_name="s")
    gather = functools.partial(
        pl.kernel,
        out_type=jax.ShapeDtypeStruct((BATCH // 128, 128), jnp.float32),
        mesh=mesh,
        scratch_types=[
            pltpu.VMEM((BW,), jnp.int32),
            pltpu.VMEM((NSTREAM, 128), jnp.float32),
            pltpu.SemaphoreType.DMA,
        ],
    )
    return gather(_sc_gather)(idx, wave)


def kernel(x, wave):
    x2 = x.reshape(BATCH, NSITES).astype(jnp.int32)
    return _run(x2, wave).reshape(x.shape[:-3])
